# baseline (speedup 1.0000x reference)
import functools

import jax
import jax.numpy as jnp
from jax import lax
from jax.experimental import pallas as pl
from jax.experimental.pallas import tpu as pltpu
from jax.experimental.pallas import tpu_sc as plsc

N = 100000
D = 128
S = 256
NC = 2
NS = 16
NW = NC * NS
SPW = S // NW
W = 256
G = D // 16
U = 8
NB = 272


def _sc_body(x_hbm, bnds_hbm, out_hbm, bnds_v, acc_v, buf0_v, buf1_v,
             sem0, sem1):
    c = lax.axis_index("c")
    s = lax.axis_index("s")
    w = s * NC + c
    seg0 = w * SPW

    pltpu.sync_copy(bnds_hbm, bnds_v)

    zero = jnp.zeros((16,), jnp.float32)
    for si in range(SPW):
        for g in range(G):
            acc_v[si, pl.ds(g * 16, 16)] = zero

    b = [bnds_v[pl.ds(seg0 + si, 16)][0] for si in range(SPW + 1)]
    r_begin = b[0]
    r_end = b[SPW]
    base0 = (r_begin // 8) * 8
    nwin = (r_end - base0 + (W - 1)) // W
    npair = (nwin + 1) // 2

    def wstart_of(k):
        return jnp.minimum(base0 + k * W, N - W)

    def start(k, buf, sem):
        @pl.when(k < nwin)
        def _():
            pltpu.async_copy(x_hbm.at[pl.ds(wstart_of(k), W)], buf, sem)

    def wait(k, buf, sem):
        @pl.when(k < nwin)
        def _():
            pltpu.make_async_copy(x_hbm.at[pl.ds(wstart_of(k), W)], buf,
                                  sem).wait()

    def process(k, buf):
        win_lo = base0 + k * W
        wstart = wstart_of(k)
        si_dyn = jnp.int32(0)
        nxt = b[1]
        for si in range(1, SPW):
            inside = b[si] <= win_lo
            si_dyn = jnp.where(inside, jnp.int32(si), si_dyn)
            nxt = jnp.where(inside, b[si + 1], nxt)
        fast = ((win_lo >= r_begin) & (win_lo == wstart)
                & (nxt >= win_lo + W))

        @pl.when(fast)
        def _():
            def rb(t, carry):
                out = list(carry)
                for r in range(U):
                    j = t * U + r
                    for g in range(G):
                        out[g] = out[g] + buf[j, pl.ds(g * 16, 16)]
                return tuple(out)

            res = lax.fori_loop(0, W // U, rb, (zero,) * G)
            for g in range(G):
                sl = pl.ds(g * 16, 16)
                acc_v[si_dyn, sl] = acc_v[si_dyn, sl] + res[g]

        @pl.when(jnp.logical_not(fast))
        def _():
            for si in range(SPW):
                a = jnp.maximum(b[si], win_lo)
                e = jnp.minimum(b[si + 1], win_lo + W)
                lo = jnp.clip(a - wstart, 0, W)
                hi = jnp.clip(e - wstart, 0, W)
                hi = jnp.maximum(hi, lo)
                n = hi - lo

                @pl.when(n > 0)
                def _():
                    def main_body(t, carry):
                        out = list(carry)
                        for r in range(U):
                            j = lo + t * U + r
                            for g in range(G):
                                out[g] = out[g] + buf[j, pl.ds(g * 16, 16)]
                        return tuple(out)

                    res = lax.fori_loop(0, n // U, main_body, (zero,) * G)

                    def row_body(j, carry):
                        return tuple(carry[g] + buf[j, pl.ds(g * 16, 16)]
                                     for g in range(G))

                    res = lax.fori_loop(lo + (n // U) * U, hi, row_body, res)
                    for g in range(G):
                        sl = pl.ds(g * 16, 16)
                        acc_v[si, sl] = acc_v[si, sl] + res[g]

    start(jnp.int32(0), buf0_v, sem0)
    start(jnp.int32(1), buf1_v, sem1)

    def pair_body(p, _):
        k0 = 2 * p
        wait(k0, buf0_v, sem0)
        process(k0, buf0_v)
        start(k0 + 2, buf0_v, sem0)
        k1 = 2 * p + 1
        wait(k1, buf1_v, sem1)
        process(k1, buf1_v)
        start(k1 + 2, buf1_v, sem1)
        return 0

    lax.fori_loop(0, npair, pair_body, 0)
    pltpu.sync_copy(acc_v, out_hbm.at[pl.ds(seg0, SPW)])


@functools.partial(
    pl.kernel,
    mesh=plsc.VectorSubcoreMesh(core_axis_name="c", subcore_axis_name="s"),
    out_type=jax.ShapeDtypeStruct((S, D), jnp.float32),
    scratch_types=[
        pltpu.VMEM((NB,), jnp.int32),
        pltpu.VMEM((SPW, D), jnp.float32),
        pltpu.VMEM((W, D), jnp.float32),
        pltpu.VMEM((W, D), jnp.float32),
        pltpu.SemaphoreType.DMA,
        pltpu.SemaphoreType.DMA,
    ],
)
def _segment_sum_sc(x_hbm, bnds_hbm, out_hbm, bnds_v, acc_v, buf0_v, buf1_v,
                    sem0, sem1):
    _sc_body(x_hbm, bnds_hbm, out_hbm, bnds_v, acc_v, buf0_v, buf1_v,
             sem0, sem1)


def kernel(x, batch):
    batch = batch.astype(jnp.int32)
    cuts = jnp.arange(S + 1, dtype=jnp.int32)
    bounds = jnp.searchsorted(batch, cuts,
                              method="compare_all").astype(jnp.int32)
    bounds = jnp.concatenate(
        [bounds, jnp.full((NB - (S + 1),), N, dtype=jnp.int32)])
    return _segment_sum_sc(x, bounds)


# speedup vs baseline: 1.2136x; 1.2136x over previous
import functools

import jax
import jax.numpy as jnp
from jax import lax
from jax.experimental import pallas as pl
from jax.experimental.pallas import tpu as pltpu
from jax.experimental.pallas import tpu_sc as plsc

N = 100000
D = 128
S = 256
NC = 2
NS = 16
NW = NC * NS
SPW = S // NW
W = 448
G = D // 16
U = 4
NB = 272


def _sc_body(x_hbm, bnds_hbm, out_hbm, bnds_v, acc_v, buf0_v, buf1_v,
             sem0, sem1):
    c = lax.axis_index("c")
    s = lax.axis_index("s")
    w = s * NC + c
    seg0 = w * SPW

    pltpu.sync_copy(bnds_hbm, bnds_v)

    zero = jnp.zeros((16,), jnp.float32)
    for si in range(SPW):
        for g in range(G):
            acc_v[si, pl.ds(g * 16, 16)] = zero

    b = [bnds_v[pl.ds(seg0 + si, 16)][0] for si in range(SPW + 1)]
    r_begin = b[0]
    r_end = b[SPW]
    base0 = (r_begin // 8) * 8
    nwin = (r_end - base0 + (W - 1)) // W
    npair = (nwin + 1) // 2

    def wstart_of(k):
        return jnp.minimum(base0 + k * W, N - W)

    def start(k, buf, sem):
        @pl.when(k < nwin)
        def _():
            pltpu.async_copy(x_hbm.at[pl.ds(wstart_of(k), W)], buf, sem)

    def wait(k, buf, sem):
        @pl.when(k < nwin)
        def _():
            pltpu.make_async_copy(x_hbm.at[pl.ds(wstart_of(k), W)], buf,
                                  sem).wait()

    def process(k, buf):
        win_lo = base0 + k * W
        wstart = wstart_of(k)
        si_dyn = jnp.int32(0)
        nxt = b[1]
        for si in range(1, SPW):
            inside = b[si] <= win_lo
            si_dyn = jnp.where(inside, jnp.int32(si), si_dyn)
            nxt = jnp.where(inside, b[si + 1], nxt)
        fast = ((win_lo >= r_begin) & (win_lo == wstart)
                & (nxt >= win_lo + W))

        @pl.when(fast)
        def _():
            def rb(t, carry):
                out = list(carry)
                for r in range(U):
                    j = t * U + r
                    for g in range(G):
                        out[g] = out[g] + buf[j, pl.ds(g * 16, 16)]
                return tuple(out)

            res = lax.fori_loop(0, W // U, rb, (zero,) * G)
            for g in range(G):
                sl = pl.ds(g * 16, 16)
                acc_v[si_dyn, sl] = acc_v[si_dyn, sl] + res[g]

        @pl.when(jnp.logical_not(fast))
        def _():
            for si in range(SPW):
                a = jnp.maximum(b[si], win_lo)
                e = jnp.minimum(b[si + 1], win_lo + W)
                lo = jnp.clip(a - wstart, 0, W)
                hi = jnp.clip(e - wstart, 0, W)
                hi = jnp.maximum(hi, lo)

                def row_body(j, carry):
                    return tuple(carry[g] + buf[j, pl.ds(g * 16, 16)]
                                 for g in range(G))

                init = tuple(acc_v[si, pl.ds(g * 16, 16)] for g in range(G))
                res = lax.fori_loop(lo, hi, row_body, init)
                for g in range(G):
                    acc_v[si, pl.ds(g * 16, 16)] = res[g]

    start(jnp.int32(0), buf0_v, sem0)
    start(jnp.int32(1), buf1_v, sem1)

    def pair_body(p, _):
        k0 = 2 * p
        wait(k0, buf0_v, sem0)
        process(k0, buf0_v)
        start(k0 + 2, buf0_v, sem0)
        k1 = 2 * p + 1
        wait(k1, buf1_v, sem1)
        process(k1, buf1_v)
        start(k1 + 2, buf1_v, sem1)
        return 0

    lax.fori_loop(0, npair, pair_body, 0)
    pltpu.sync_copy(acc_v, out_hbm.at[pl.ds(seg0, SPW)])


@functools.partial(
    pl.kernel,
    mesh=plsc.VectorSubcoreMesh(core_axis_name="c", subcore_axis_name="s"),
    out_type=jax.ShapeDtypeStruct((S, D), jnp.float32),
    scratch_types=[
        pltpu.VMEM((NB,), jnp.int32),
        pltpu.VMEM((SPW, D), jnp.float32),
        pltpu.VMEM((W, D), jnp.float32),
        pltpu.VMEM((W, D), jnp.float32),
        pltpu.SemaphoreType.DMA,
        pltpu.SemaphoreType.DMA,
    ],
)
def _segment_sum_sc(x_hbm, bnds_hbm, out_hbm, bnds_v, acc_v, buf0_v, buf1_v,
                    sem0, sem1):
    _sc_body(x_hbm, bnds_hbm, out_hbm, bnds_v, acc_v, buf0_v, buf1_v,
             sem0, sem1)


def kernel(x, batch):
    batch = batch.astype(jnp.int32)
    cuts = jnp.arange(S + 1, dtype=jnp.int32)
    bounds = jnp.searchsorted(batch, cuts,
                              method="compare_all").astype(jnp.int32)
    bounds = jnp.concatenate(
        [bounds, jnp.full((NB - (S + 1),), N, dtype=jnp.int32)])
    return _segment_sum_sc(x, bounds)


# speedup vs baseline: 1.2378x; 1.0199x over previous
import functools

import jax
import jax.numpy as jnp
from jax import lax
from jax.experimental import pallas as pl
from jax.experimental.pallas import tpu as pltpu
from jax.experimental.pallas import tpu_sc as plsc

N = 100000
D = 128
S = 256
NC = 2
NS = 16
NW = NC * NS
SPW = S // NW
W = 448
G = D // 16
U = 4
BS_STEPS = 17


def _sc_body(x_hbm, batch_hbm, out_hbm, acc_v, buf0_v, buf1_v, vals_v,
             sem0, sem1):
    c = lax.axis_index("c")
    s = lax.axis_index("s")
    w = s * NC + c
    seg0 = w * SPW

    zero = jnp.zeros((16,), jnp.float32)
    for si in range(SPW):
        for g in range(G):
            acc_v[si, pl.ds(g * 16, 16)] = zero

    cuts = seg0 + lax.iota(jnp.int32, 16)
    lo_v = jnp.zeros((16,), jnp.int32)
    hi_v = jnp.full((16,), N, jnp.int32)
    for _ in range(BS_STEPS):
        mid = lax.div(lo_v + hi_v, 2)
        midc = jnp.minimum(mid, N - 1)
        pltpu.async_copy(batch_hbm.at[midc], vals_v, sem0).wait()
        pred = vals_v[...] < cuts
        hi_v = jnp.where(pred, hi_v, mid)
        lo_v = jnp.minimum(jnp.where(pred, mid + 1, lo_v), hi_v)

    b = [lo_v[si] for si in range(SPW + 1)]
    r_begin = b[0]
    r_end = b[SPW]
    base0 = (r_begin // 8) * 8
    nwin = (r_end - base0 + (W - 1)) // W
    npair = (nwin + 1) // 2

    def wstart_of(k):
        return jnp.minimum(base0 + k * W, N - W)

    def start(k, buf, sem):
        @pl.when(k < nwin)
        def _():
            pltpu.async_copy(x_hbm.at[pl.ds(wstart_of(k), W)], buf, sem)

    def wait(k, buf, sem):
        @pl.when(k < nwin)
        def _():
            pltpu.make_async_copy(x_hbm.at[pl.ds(wstart_of(k), W)], buf,
                                  sem).wait()

    def process(k, buf):
        win_lo = base0 + k * W
        wstart = wstart_of(k)
        si_dyn = jnp.int32(0)
        nxt = b[1]
        for si in range(1, SPW):
            inside = b[si] <= win_lo
            si_dyn = jnp.where(inside, jnp.int32(si), si_dyn)
            nxt = jnp.where(inside, b[si + 1], nxt)
        fast = ((win_lo >= r_begin) & (win_lo == wstart)
                & (nxt >= win_lo + W))

        @pl.when(fast)
        def _():
            def rb(t, carry):
                out = list(carry)
                for r in range(U):
                    j = t * U + r
                    for g in range(G):
                        out[g] = out[g] + buf[j, pl.ds(g * 16, 16)]
                return tuple(out)

            res = lax.fori_loop(0, W // U, rb, (zero,) * G)
            for g in range(G):
                sl = pl.ds(g * 16, 16)
                acc_v[si_dyn, sl] = acc_v[si_dyn, sl] + res[g]

        @pl.when(jnp.logical_not(fast))
        def _():
            for si in range(SPW):
                a = jnp.maximum(b[si], win_lo)
                e = jnp.minimum(b[si + 1], win_lo + W)
                lo = jnp.clip(a - wstart, 0, W)
                hi = jnp.clip(e - wstart, 0, W)
                hi = jnp.maximum(hi, lo)

                def row_body(j, carry):
                    return tuple(carry[g] + buf[j, pl.ds(g * 16, 16)]
                                 for g in range(G))

                init = tuple(acc_v[si, pl.ds(g * 16, 16)] for g in range(G))
                res = lax.fori_loop(lo, hi, row_body, init)
                for g in range(G):
                    acc_v[si, pl.ds(g * 16, 16)] = res[g]

    start(jnp.int32(0), buf0_v, sem0)
    start(jnp.int32(1), buf1_v, sem1)

    def pair_body(p, _):
        k0 = 2 * p
        wait(k0, buf0_v, sem0)
        process(k0, buf0_v)
        start(k0 + 2, buf0_v, sem0)
        k1 = 2 * p + 1
        wait(k1, buf1_v, sem1)
        process(k1, buf1_v)
        start(k1 + 2, buf1_v, sem1)
        return 0

    lax.fori_loop(0, npair, pair_body, 0)
    pltpu.sync_copy(acc_v, out_hbm.at[pl.ds(seg0, SPW)])


@functools.partial(
    pl.kernel,
    mesh=plsc.VectorSubcoreMesh(core_axis_name="c", subcore_axis_name="s"),
    out_type=jax.ShapeDtypeStruct((S, D), jnp.float32),
    scratch_types=[
        pltpu.VMEM((SPW, D), jnp.float32),
        pltpu.VMEM((W, D), jnp.float32),
        pltpu.VMEM((W, D), jnp.float32),
        pltpu.VMEM((16,), jnp.int32),
        pltpu.SemaphoreType.DMA,
        pltpu.SemaphoreType.DMA,
    ],
)
def _segment_sum_sc(x_hbm, batch_hbm, out_hbm, acc_v, buf0_v, buf1_v, vals_v,
                    sem0, sem1):
    _sc_body(x_hbm, batch_hbm, out_hbm, acc_v, buf0_v, buf1_v, vals_v,
             sem0, sem1)


def kernel(x, batch):
    return _segment_sum_sc(x, batch.astype(jnp.int32))


# speedup vs baseline: 1.4555x; 1.1759x over previous
import functools

import jax
import jax.numpy as jnp
from jax import lax
from jax.experimental import pallas as pl
from jax.experimental.pallas import tpu as pltpu
from jax.experimental.pallas import tpu_sc as plsc

N = 100000
D = 128
S = 256
NC = 2
NS = 16
NW = NC * NS
SPW = S // NW
W = 448
G = D // 16
U = 4
BS_STEPS = 17


def _sc_body(x_hbm, batch_hbm, out_hbm, acc_v, buf0_v, buf1_v, vals_v,
             batch_sp, sem0, sem1):
    c = lax.axis_index("c")
    s = lax.axis_index("s")
    w = s * NC + c
    seg0 = w * SPW

    @pl.when(s == 0)
    def _():
        pltpu.sync_copy(batch_hbm, batch_sp)

    zero = jnp.zeros((16,), jnp.float32)
    for si in range(SPW):
        for g in range(G):
            acc_v[si, pl.ds(g * 16, 16)] = zero

    plsc.subcore_barrier()

    cuts = seg0 + lax.iota(jnp.int32, 16)
    lo_v = jnp.zeros((16,), jnp.int32)
    hi_v = jnp.full((16,), N, jnp.int32)
    for _ in range(BS_STEPS):
        mid = lax.div(lo_v + hi_v, 2)
        midc = jnp.minimum(mid, N - 1)
        pltpu.async_copy(batch_sp.at[midc], vals_v, sem0).wait()
        pred = vals_v[...] < cuts
        hi_v = jnp.where(pred, hi_v, mid)
        lo_v = jnp.minimum(jnp.where(pred, mid + 1, lo_v), hi_v)

    b = [lo_v[si] for si in range(SPW + 1)]
    r_begin = b[0]
    r_end = b[SPW]
    base0 = (r_begin // 8) * 8
    nwin = (r_end - base0 + (W - 1)) // W
    npair = (nwin + 1) // 2

    def wstart_of(k):
        return jnp.minimum(base0 + k * W, N - W)

    def start(k, buf, sem):
        @pl.when(k < nwin)
        def _():
            pltpu.async_copy(x_hbm.at[pl.ds(wstart_of(k), W)], buf, sem)

    def wait(k, buf, sem):
        @pl.when(k < nwin)
        def _():
            pltpu.make_async_copy(x_hbm.at[pl.ds(wstart_of(k), W)], buf,
                                  sem).wait()

    def process(k, buf):
        win_lo = base0 + k * W
        wstart = wstart_of(k)
        si_dyn = jnp.int32(0)
        nxt = b[1]
        for si in range(1, SPW):
            inside = b[si] <= win_lo
            si_dyn = jnp.where(inside, jnp.int32(si), si_dyn)
            nxt = jnp.where(inside, b[si + 1], nxt)
        fast = ((win_lo >= r_begin) & (win_lo == wstart)
                & (nxt >= win_lo + W))

        @pl.when(fast)
        def _():
            def rb(t, carry):
                out = list(carry)
                for r in range(U):
                    j = t * U + r
                    for g in range(G):
                        out[g] = out[g] + buf[j, pl.ds(g * 16, 16)]
                return tuple(out)

            res = lax.fori_loop(0, W // U, rb, (zero,) * G)
            for g in range(G):
                sl = pl.ds(g * 16, 16)
                acc_v[si_dyn, sl] = acc_v[si_dyn, sl] + res[g]

        @pl.when(jnp.logical_not(fast))
        def _():
            for si in range(SPW):
                a = jnp.maximum(b[si], win_lo)
                e = jnp.minimum(b[si + 1], win_lo + W)
                lo = jnp.clip(a - wstart, 0, W)
                hi = jnp.clip(e - wstart, 0, W)
                hi = jnp.maximum(hi, lo)

                def row_body(j, carry):
                    return tuple(carry[g] + buf[j, pl.ds(g * 16, 16)]
                                 for g in range(G))

                init = tuple(acc_v[si, pl.ds(g * 16, 16)] for g in range(G))
                res = lax.fori_loop(lo, hi, row_body, init)
                for g in range(G):
                    acc_v[si, pl.ds(g * 16, 16)] = res[g]

    start(jnp.int32(0), buf0_v, sem0)
    start(jnp.int32(1), buf1_v, sem1)

    def pair_body(p, _):
        k0 = 2 * p
        wait(k0, buf0_v, sem0)
        process(k0, buf0_v)
        start(k0 + 2, buf0_v, sem0)
        k1 = 2 * p + 1
        wait(k1, buf1_v, sem1)
        process(k1, buf1_v)
        start(k1 + 2, buf1_v, sem1)
        return 0

    lax.fori_loop(0, npair, pair_body, 0)
    pltpu.sync_copy(acc_v, out_hbm.at[pl.ds(seg0, SPW)])


@functools.partial(
    pl.kernel,
    mesh=plsc.VectorSubcoreMesh(core_axis_name="c", subcore_axis_name="s"),
    out_type=jax.ShapeDtypeStruct((S, D), jnp.float32),
    scratch_types=[
        pltpu.VMEM((SPW, D), jnp.float32),
        pltpu.VMEM((W, D), jnp.float32),
        pltpu.VMEM((W, D), jnp.float32),
        pltpu.VMEM((16,), jnp.int32),
        pltpu.VMEM_SHARED((N,), jnp.int32),
        pltpu.SemaphoreType.DMA,
        pltpu.SemaphoreType.DMA,
    ],
)
def _segment_sum_sc(x_hbm, batch_hbm, out_hbm, acc_v, buf0_v, buf1_v, vals_v,
                    batch_sp, sem0, sem1):
    _sc_body(x_hbm, batch_hbm, out_hbm, acc_v, buf0_v, buf1_v, vals_v,
             batch_sp, sem0, sem1)


def kernel(x, batch):
    return _segment_sum_sc(x, batch.astype(jnp.int32))


# speedup vs baseline: 1.4671x; 1.0080x over previous
import functools

import jax
import jax.numpy as jnp
from jax import lax
from jax.experimental import pallas as pl
from jax.experimental.pallas import tpu as pltpu
from jax.experimental.pallas import tpu_sc as plsc

N = 100000
D = 128
S = 256
NC = 2
NS = 16
NW = NC * NS
SPW = S // NW
W = 320
G = D // 16
U = 4
BS_STEPS = 17


def _sc_body(x_hbm, batch_hbm, out_hbm, acc_v, buf0_v, buf1_v, buf2_v,
             vals_v, batch_sp, sem0, sem1, sem2):
    c = lax.axis_index("c")
    s = lax.axis_index("s")
    w = s * NC + c
    seg0 = w * SPW

    @pl.when(s == 0)
    def _():
        pltpu.sync_copy(batch_hbm, batch_sp)

    zero = jnp.zeros((16,), jnp.float32)
    for si in range(SPW):
        for g in range(G):
            acc_v[si, pl.ds(g * 16, 16)] = zero

    plsc.subcore_barrier()

    cuts = seg0 + lax.iota(jnp.int32, 16)
    lo_v = jnp.zeros((16,), jnp.int32)
    hi_v = jnp.full((16,), N, jnp.int32)
    for _ in range(BS_STEPS):
        mid = lax.div(lo_v + hi_v, 2)
        midc = jnp.minimum(mid, N - 1)
        pltpu.async_copy(batch_sp.at[midc], vals_v, sem0).wait()
        pred = vals_v[...] < cuts
        hi_v = jnp.where(pred, hi_v, mid)
        lo_v = jnp.minimum(jnp.where(pred, mid + 1, lo_v), hi_v)

    b = [lo_v[si] for si in range(SPW + 1)]
    r_begin = b[0]
    r_end = b[SPW]
    base0 = (r_begin // 8) * 8
    nwin = (r_end - base0 + (W - 1)) // W
    ntrip = (nwin + 2) // 3

    def wstart_of(k):
        return jnp.minimum(base0 + k * W, N - W)

    def start(k, buf, sem):
        @pl.when(k < nwin)
        def _():
            pltpu.async_copy(x_hbm.at[pl.ds(wstart_of(k), W)], buf, sem)

    def wait(k, buf, sem):
        @pl.when(k < nwin)
        def _():
            pltpu.make_async_copy(x_hbm.at[pl.ds(wstart_of(k), W)], buf,
                                  sem).wait()

    def process(k, buf):
        win_lo = base0 + k * W
        wstart = wstart_of(k)
        si_dyn = jnp.int32(0)
        nxt = b[1]
        for si in range(1, SPW):
            inside = b[si] <= win_lo
            si_dyn = jnp.where(inside, jnp.int32(si), si_dyn)
            nxt = jnp.where(inside, b[si + 1], nxt)
        fast = ((win_lo >= r_begin) & (win_lo == wstart)
                & (nxt >= win_lo + W))

        @pl.when(fast)
        def _():
            def rb(t, carry):
                out = list(carry)
                for r in range(U):
                    j = t * U + r
                    for g in range(G):
                        out[g] = out[g] + buf[j, pl.ds(g * 16, 16)]
                return tuple(out)

            res = lax.fori_loop(0, W // U, rb, (zero,) * G)
            for g in range(G):
                sl = pl.ds(g * 16, 16)
                acc_v[si_dyn, sl] = acc_v[si_dyn, sl] + res[g]

        @pl.when(jnp.logical_not(fast))
        def _():
            for si in range(SPW):
                a = jnp.maximum(b[si], win_lo)
                e = jnp.minimum(b[si + 1], win_lo + W)
                lo = jnp.clip(a - wstart, 0, W)
                hi = jnp.clip(e - wstart, 0, W)
                hi = jnp.maximum(hi, lo)

                def row_body(j, carry):
                    return tuple(carry[g] + buf[j, pl.ds(g * 16, 16)]
                                 for g in range(G))

                init = tuple(acc_v[si, pl.ds(g * 16, 16)] for g in range(G))
                res = lax.fori_loop(lo, hi, row_body, init)
                for g in range(G):
                    acc_v[si, pl.ds(g * 16, 16)] = res[g]

    start(jnp.int32(0), buf0_v, sem0)
    start(jnp.int32(1), buf1_v, sem1)
    start(jnp.int32(2), buf2_v, sem2)

    def trip_body(p, _):
        for ph, (buf, sem) in enumerate(
                ((buf0_v, sem0), (buf1_v, sem1), (buf2_v, sem2))):
            k = 3 * p + ph
            wait(k, buf, sem)
            process(k, buf)
            start(k + 3, buf, sem)
        return 0

    lax.fori_loop(0, ntrip, trip_body, 0)
    pltpu.sync_copy(acc_v, out_hbm.at[pl.ds(seg0, SPW)])


@functools.partial(
    pl.kernel,
    mesh=plsc.VectorSubcoreMesh(core_axis_name="c", subcore_axis_name="s"),
    out_type=jax.ShapeDtypeStruct((S, D), jnp.float32),
    scratch_types=[
        pltpu.VMEM((SPW, D), jnp.float32),
        pltpu.VMEM((W, D), jnp.float32),
        pltpu.VMEM((W, D), jnp.float32),
        pltpu.VMEM((W, D), jnp.float32),
        pltpu.VMEM((16,), jnp.int32),
        pltpu.VMEM_SHARED((N,), jnp.int32),
        pltpu.SemaphoreType.DMA,
        pltpu.SemaphoreType.DMA,
        pltpu.SemaphoreType.DMA,
    ],
)
def _segment_sum_sc(x_hbm, batch_hbm, out_hbm, acc_v, buf0_v, buf1_v, buf2_v,
                    vals_v, batch_sp, sem0, sem1, sem2):
    _sc_body(x_hbm, batch_hbm, out_hbm, acc_v, buf0_v, buf1_v, buf2_v,
             vals_v, batch_sp, sem0, sem1, sem2)


def kernel(x, batch):
    return _segment_sum_sc(x, batch.astype(jnp.int32))
